# baseline (speedup 1.0000x reference)
import functools
import math

import jax
import jax.numpy as jnp
from jax.experimental import pallas as pl

N_NODES = 1024
HID = 64
MAXSTEP = 2


def _att_kernel(state_ref, enc_ref, out_ref, *, t_len, scale):
    st = state_ref[0]
    es = []
    for t in range(t_len):
        e = jnp.sum(st * enc_ref[0, t], axis=1, keepdims=True)
        es.append(e * (1.0 / scale))
    mx = es[0]
    for t in range(1, t_len):
        mx = jnp.maximum(mx, es[t])
    ws = [jnp.exp(e - mx) for e in es]
    den = ws[0]
    for t in range(1, t_len):
        den = den + ws[t]
    acc = ws[0] * enc_ref[0, 0]
    for t in range(1, t_len):
        acc = acc + ws[t] * enc_ref[0, t]
    out_ref[0] = acc / den


def _diff_kernel(s_ref, x_ref, d_ref):
    s = s_ref[0]
    x1 = jnp.dot(s, x_ref[...], preferred_element_type=jnp.float32)
    x2 = jnp.dot(s, x1, preferred_element_type=jnp.float32)
    d_ref[0, 0] = x1
    d_ref[0, 1] = x2


def _gate_kernel(inp_ref, wt_ref, st_ref, dir_ref, dwr_ref, dsr_ref,
                 wg_ref, bg_ref, rs_ref, u_ref):
    cols = [inp_ref[0], wt_ref[0], st_ref[0]]
    for s in range(2):
        for k in range(2):
            cols += [dir_ref[0, s, k], dwr_ref[0, s, k], dsr_ref[0, s, k]]
    x = jnp.concatenate(cols, axis=1)
    g = jnp.dot(x, wg_ref[...], preferred_element_type=jnp.float32) + bg_ref[...]
    g = jax.nn.sigmoid(g)
    r = g[:, :HID]
    u = g[:, HID:]
    rs_ref[0] = r * st_ref[0]
    u_ref[0] = u


def _final_kernel(inp_ref, wt_ref, rs_ref, dir_ref, dwr_ref, drr_ref,
                  u_ref, st_ref, wc_ref, bc_ref, p1_ref, p1b_ref,
                  p2_ref, p2b_ref, o_ref):
    cols = [inp_ref[0], wt_ref[0], rs_ref[0]]
    for s in range(2):
        for k in range(2):
            cols += [dir_ref[0, s, k], dwr_ref[0, s, k], drr_ref[0, s, k]]
    x = jnp.concatenate(cols, axis=1)
    c = jnp.dot(x, wc_ref[...], preferred_element_type=jnp.float32) + bc_ref[...]
    c = jnp.tanh(c)
    u = u_ref[0]
    ns = u * st_ref[0] + (1.0 - u) * c
    h = jnp.dot(ns, p1_ref[...], preferred_element_type=jnp.float32) + p1b_ref[...]
    h = jnp.maximum(h, 0.0)
    o_ref[0] = jnp.dot(h, p2_ref[...], preferred_element_type=jnp.float32) + p2b_ref[...]


def _diffuse(s2, x, n_tiles):
    n, c = x.shape
    ct = c // n_tiles
    return pl.pallas_call(
        _diff_kernel,
        grid=(2, n_tiles),
        in_specs=[
            pl.BlockSpec((1, n, n), lambda s, cc: (s, 0, 0)),
            pl.BlockSpec((n, ct), lambda s, cc: (0, cc)),
        ],
        out_specs=pl.BlockSpec((1, 2, n, ct), lambda s, cc: (s, 0, 0, cc)),
        out_shape=jax.ShapeDtypeStruct((2, 2, n, c), jnp.float32),
    )(s2, x)


def kernel(support0, support1, inputs, state, encoder_outputs,
           W_gate, b_gate, W_cand, b_cand, P1, p1b, P2, p2b):
    b, n, in_dim = inputs.shape
    t_len = encoder_outputs.shape[1]
    h = state.shape[2]
    scale = math.sqrt(float(n * h))

    s2 = jnp.stack([support0, support1])

    weighted = pl.pallas_call(
        functools.partial(_att_kernel, t_len=t_len, scale=scale),
        grid=(b,),
        in_specs=[
            pl.BlockSpec((1, n, h), lambda i: (i, 0, 0)),
            pl.BlockSpec((1, t_len, n, h), lambda i: (i, 0, 0, 0)),
        ],
        out_specs=pl.BlockSpec((1, n, h), lambda i: (i, 0, 0)),
        out_shape=jax.ShapeDtypeStruct((b, n, h), jnp.float32),
    )(state, encoder_outputs)

    i0 = inputs.transpose(1, 0, 2).reshape(n, b * in_dim)
    wt = weighted.transpose(1, 0, 2).reshape(n, b * h)
    st = state.transpose(1, 0, 2).reshape(n, b * h)

    di = _diffuse(s2, i0, 1)
    dw = _diffuse(s2, wt, 2)
    ds = _diffuse(s2, st, 2)

    def per_b(d, f):
        return d.reshape(2, 2, n, b, f).transpose(3, 0, 1, 2, 4)

    dir_ = per_b(di, in_dim)
    dwr = per_b(dw, h)
    dsr = per_b(ds, h)

    in_size = in_dim + 2 * h
    nm = 2 * MAXSTEP + 1
    wg2 = W_gate.reshape(in_size, nm, 2 * h).transpose(1, 0, 2).reshape(in_size * nm, 2 * h)
    wc2 = W_cand.reshape(in_size, nm, h).transpose(1, 0, 2).reshape(in_size * nm, h)

    full = lambda shp: pl.BlockSpec(shp, lambda i: tuple(0 for _ in shp))
    bspec = lambda *shp: pl.BlockSpec((1,) + tuple(shp), lambda i: (i,) + tuple(0 for _ in shp))

    rs, u = pl.pallas_call(
        _gate_kernel,
        grid=(b,),
        in_specs=[
            bspec(n, in_dim), bspec(n, h), bspec(n, h),
            bspec(2, 2, n, in_dim), bspec(2, 2, n, h), bspec(2, 2, n, h),
            full((in_size * nm, 2 * h)), full((1, 2 * h)),
        ],
        out_specs=[bspec(n, h), bspec(n, h)],
        out_shape=[jax.ShapeDtypeStruct((b, n, h), jnp.float32),
                   jax.ShapeDtypeStruct((b, n, h), jnp.float32)],
    )(inputs, weighted, state, dir_, dwr, dsr, wg2, b_gate.reshape(1, 2 * h))

    rst = rs.transpose(1, 0, 2).reshape(n, b * h)
    dr = _diffuse(s2, rst, 2)
    drr = per_b(dr, h)

    out = pl.pallas_call(
        _final_kernel,
        grid=(b,),
        in_specs=[
            bspec(n, in_dim), bspec(n, h), bspec(n, h),
            bspec(2, 2, n, in_dim), bspec(2, 2, n, h), bspec(2, 2, n, h),
            bspec(n, h), bspec(n, h),
            full((in_size * nm, h)), full((1, h)),
            full((h, h)), full((1, h)), full((h, 1)), full((1, 1)),
        ],
        out_specs=bspec(n, 1),
        out_shape=jax.ShapeDtypeStruct((b, n, 1), jnp.float32),
    )(inputs, weighted, rs, dir_, dwr, drr, u, state, wc2,
      b_cand.reshape(1, h), P1, p1b.reshape(1, h), P2, p2b.reshape(1, 1))

    return out.reshape(b, n)


# speedup vs baseline: 1.6172x; 1.6172x over previous
import functools
import math

import jax
import jax.numpy as jnp
from jax.experimental import pallas as pl

HID = 64
MAXSTEP = 2
ATT_BB = 2
PROJ_BB = 4


def _att_kernel(state_ref, enc_ref, out_ref, *, t_len, scale):
    for j in range(ATT_BB):
        st = state_ref[j]
        es = []
        for t in range(t_len):
            e = jnp.sum(st * enc_ref[j, t], axis=1, keepdims=True)
            es.append(e * (1.0 / scale))
        mx = es[0]
        for t in range(1, t_len):
            mx = jnp.maximum(mx, es[t])
        ws = [jnp.exp(e - mx) for e in es]
        den = ws[0]
        for t in range(1, t_len):
            den = den + ws[t]
        acc = ws[0] * enc_ref[j, 0]
        for t in range(1, t_len):
            acc = acc + ws[t] * enc_ref[j, t]
        out_ref[:, j * HID:(j + 1) * HID] = acc / den


def _diff_kernel(s_ref, x_ref, d_ref):
    s = s_ref[0]
    x1 = jnp.dot(s, x_ref[...], preferred_element_type=jnp.float32)
    x2 = jnp.dot(s, x1, preferred_element_type=jnp.float32)
    d_ref[0, 0] = x1
    d_ref[0, 1] = x2


def _pieces(i0_ref, di_ref, wt_ref, dw_ref, st_ref, ds_ref, j, in_dim):
    cols = [i0_ref[0][:, j * in_dim:(j + 1) * in_dim],
            wt_ref[:, j * HID:(j + 1) * HID],
            st_ref[:, j * HID:(j + 1) * HID]]
    for s in range(2):
        for k in range(2):
            cols += [di_ref[0, s, k][:, j * in_dim:(j + 1) * in_dim],
                     dw_ref[s, k][:, j * HID:(j + 1) * HID],
                     ds_ref[s, k][:, j * HID:(j + 1) * HID]]
    return jnp.concatenate(cols, axis=1)


def _gate_kernel(i0_ref, di_ref, wt_ref, dw_ref, st_ref, ds_ref,
                 wg_ref, bg_ref, rst_ref, u_ref, *, in_dim, n):
    xs = [_pieces(i0_ref, di_ref, wt_ref, dw_ref, st_ref, ds_ref, j, in_dim)
          for j in range(PROJ_BB)]
    x = jnp.concatenate(xs, axis=0)
    g = jnp.dot(x, wg_ref[...], preferred_element_type=jnp.float32) + bg_ref[...]
    g = jax.nn.sigmoid(g)
    for j in range(PROJ_BB):
        gj = g[j * n:(j + 1) * n]
        stj = st_ref[:, j * HID:(j + 1) * HID]
        rst_ref[:, j * HID:(j + 1) * HID] = gj[:, :HID] * stj
        u_ref[j] = gj[:, HID:]


def _final_kernel(i0_ref, di_ref, wt_ref, dw_ref, rst_ref, dr_ref,
                  u_ref, st_ref, wc_ref, bc_ref, p1_ref, p1b_ref,
                  p2_ref, p2b_ref, o_ref, *, in_dim, n):
    xs = [_pieces(i0_ref, di_ref, wt_ref, dw_ref, rst_ref, dr_ref, j, in_dim)
          for j in range(PROJ_BB)]
    x = jnp.concatenate(xs, axis=0)
    c = jnp.dot(x, wc_ref[...], preferred_element_type=jnp.float32) + bc_ref[...]
    c = jnp.tanh(c)
    us = jnp.concatenate([u_ref[j] for j in range(PROJ_BB)], axis=0)
    sts = jnp.concatenate(
        [st_ref[:, j * HID:(j + 1) * HID] for j in range(PROJ_BB)], axis=0)
    ns = us * sts + (1.0 - us) * c
    h = jnp.dot(ns, p1_ref[...], preferred_element_type=jnp.float32) + p1b_ref[...]
    h = jnp.maximum(h, 0.0)
    o = jnp.dot(h, p2_ref[...], preferred_element_type=jnp.float32) + p2b_ref[...]
    for j in range(PROJ_BB):
        o_ref[j] = o[j * n:(j + 1) * n]


def _diffuse(s2, x, n_tiles):
    n, c = x.shape
    ct = c // n_tiles
    return pl.pallas_call(
        _diff_kernel,
        grid=(2, n_tiles),
        in_specs=[
            pl.BlockSpec((1, n, n), lambda s, cc: (s, 0, 0)),
            pl.BlockSpec((n, ct), lambda s, cc: (0, cc)),
        ],
        out_specs=pl.BlockSpec((1, 2, n, ct), lambda s, cc: (s, 0, 0, cc)),
        out_shape=jax.ShapeDtypeStruct((2, 2, n, c), jnp.float32),
    )(s2, x)


def kernel(support0, support1, inputs, state, encoder_outputs,
           W_gate, b_gate, W_cand, b_cand, P1, p1b, P2, p2b):
    b, n, in_dim = inputs.shape
    t_len = encoder_outputs.shape[1]
    h = state.shape[2]
    scale = math.sqrt(float(n * h))
    n_chunks = b // PROJ_BB

    s2 = jnp.stack([support0, support1])

    wt = pl.pallas_call(
        functools.partial(_att_kernel, t_len=t_len, scale=scale),
        grid=(b // ATT_BB,),
        in_specs=[
            pl.BlockSpec((ATT_BB, n, h), lambda i: (i, 0, 0)),
            pl.BlockSpec((ATT_BB, t_len, n, h), lambda i: (i, 0, 0, 0)),
        ],
        out_specs=pl.BlockSpec((n, ATT_BB * h), lambda i: (0, i)),
        out_shape=jax.ShapeDtypeStruct((n, b * h), jnp.float32),
    )(state, encoder_outputs)

    st = state.transpose(1, 0, 2).reshape(n, b * h)
    i0 = inputs.transpose(1, 0, 2).reshape(n, b * in_dim)

    di = _diffuse(s2, i0, 1)
    dw = _diffuse(s2, wt, 2)
    ds = _diffuse(s2, st, 2)

    cin = PROJ_BB * in_dim
    i0r = i0.reshape(n, n_chunks, cin).transpose(1, 0, 2)
    dir2 = di.reshape(2, 2, n, n_chunks, cin).transpose(3, 0, 1, 2, 4)

    in_size = in_dim + 2 * h
    nm = 2 * MAXSTEP + 1
    wg2 = W_gate.reshape(in_size, nm, 2 * h).transpose(1, 0, 2).reshape(in_size * nm, 2 * h)
    wc2 = W_cand.reshape(in_size, nm, h).transpose(1, 0, 2).reshape(in_size * nm, h)

    full = lambda shp: pl.BlockSpec(shp, lambda i: tuple(0 for _ in shp))
    cw = PROJ_BB * h
    i0_spec = pl.BlockSpec((1, n, cin), lambda i: (i, 0, 0))
    di_spec = pl.BlockSpec((1, 2, 2, n, cin), lambda i: (i, 0, 0, 0, 0))
    col_spec = pl.BlockSpec((n, cw), lambda i: (0, i))
    d_spec = pl.BlockSpec((2, 2, n, cw), lambda i: (0, 0, 0, i))
    u_spec = pl.BlockSpec((PROJ_BB, n, h), lambda i: (i, 0, 0))

    rst, u = pl.pallas_call(
        functools.partial(_gate_kernel, in_dim=in_dim, n=n),
        grid=(n_chunks,),
        in_specs=[i0_spec, di_spec, col_spec, d_spec, col_spec, d_spec,
                  full((in_size * nm, 2 * h)), full((1, 2 * h))],
        out_specs=[col_spec, u_spec],
        out_shape=[jax.ShapeDtypeStruct((n, b * h), jnp.float32),
                   jax.ShapeDtypeStruct((b, n, h), jnp.float32)],
    )(i0r, dir2, wt, dw, st, ds, wg2, b_gate.reshape(1, 2 * h))

    dr = _diffuse(s2, rst, 2)

    out = pl.pallas_call(
        functools.partial(_final_kernel, in_dim=in_dim, n=n),
        grid=(n_chunks,),
        in_specs=[i0_spec, di_spec, col_spec, d_spec, col_spec, d_spec,
                  u_spec, col_spec,
                  full((in_size * nm, h)), full((1, h)),
                  full((h, h)), full((1, h)), full((h, 1)), full((1, 1))],
        out_specs=pl.BlockSpec((PROJ_BB, n, 1), lambda i: (i, 0, 0)),
        out_shape=jax.ShapeDtypeStruct((b, n, 1), jnp.float32),
    )(i0r, dir2, wt, dw, rst, dr, u, st, wc2,
      b_cand.reshape(1, h), P1, p1b.reshape(1, h), P2, p2b.reshape(1, 1))

    return out.reshape(b, n)


# speedup vs baseline: 1.6424x; 1.0156x over previous
import functools
import math

import jax
import jax.numpy as jnp
from jax.experimental import pallas as pl

HID = 64
MAXSTEP = 2
ATT_BB = 2
PROJ_BB = 4


def _att_kernel(state_ref, enc_ref, out_ref, *, t_len, scale):
    for j in range(ATT_BB):
        st = state_ref[j]
        es = []
        for t in range(t_len):
            e = jnp.sum(st * enc_ref[j, t], axis=1, keepdims=True)
            es.append(e * (1.0 / scale))
        mx = es[0]
        for t in range(1, t_len):
            mx = jnp.maximum(mx, es[t])
        ws = [jnp.exp(e - mx) for e in es]
        den = ws[0]
        for t in range(1, t_len):
            den = den + ws[t]
        acc = ws[0] * enc_ref[j, 0]
        for t in range(1, t_len):
            acc = acc + ws[t] * enc_ref[j, t]
        out_ref[:, j * HID:(j + 1) * HID] = acc / den


def _diff_kernel(s_ref, x_ref, d_ref):
    s = s_ref[0]
    x = x_ref[...].astype(jnp.bfloat16)
    x1 = jnp.dot(s, x, preferred_element_type=jnp.float32)
    x2 = jnp.dot(s, x1.astype(jnp.bfloat16), preferred_element_type=jnp.float32)
    d_ref[0, 0] = x1
    d_ref[0, 1] = x2


def _pieces(i0_ref, di_ref, wt_ref, dw_ref, st_ref, ds_ref, j, in_dim):
    cols = [i0_ref[0][:, j * in_dim:(j + 1) * in_dim],
            wt_ref[:, j * HID:(j + 1) * HID],
            st_ref[:, j * HID:(j + 1) * HID]]
    for s in range(2):
        for k in range(2):
            cols += [di_ref[0, s, k][:, j * in_dim:(j + 1) * in_dim],
                     dw_ref[s, k][:, j * HID:(j + 1) * HID],
                     ds_ref[s, k][:, j * HID:(j + 1) * HID]]
    return jnp.concatenate(cols, axis=1)


def _gate_kernel(i0_ref, di_ref, wt_ref, dw_ref, st_ref, ds_ref,
                 wg_ref, bg_ref, rst_ref, u_ref, *, in_dim, n):
    xs = [_pieces(i0_ref, di_ref, wt_ref, dw_ref, st_ref, ds_ref, j, in_dim)
          for j in range(PROJ_BB)]
    x = jnp.concatenate(xs, axis=0)
    g = jnp.dot(x, wg_ref[...], preferred_element_type=jnp.float32) + bg_ref[...]
    g = jax.nn.sigmoid(g)
    for j in range(PROJ_BB):
        gj = g[j * n:(j + 1) * n]
        stj = st_ref[:, j * HID:(j + 1) * HID]
        rst_ref[:, j * HID:(j + 1) * HID] = gj[:, :HID] * stj
        u_ref[j] = gj[:, HID:]


def _final_kernel(i0_ref, di_ref, wt_ref, dw_ref, rst_ref, dr_ref,
                  u_ref, st_ref, wc_ref, bc_ref, p1_ref, p1b_ref,
                  p2_ref, p2b_ref, o_ref, *, in_dim, n):
    xs = [_pieces(i0_ref, di_ref, wt_ref, dw_ref, rst_ref, dr_ref, j, in_dim)
          for j in range(PROJ_BB)]
    x = jnp.concatenate(xs, axis=0)
    c = jnp.dot(x, wc_ref[...], preferred_element_type=jnp.float32) + bc_ref[...]
    c = jnp.tanh(c)
    us = jnp.concatenate([u_ref[j] for j in range(PROJ_BB)], axis=0)
    sts = jnp.concatenate(
        [st_ref[:, j * HID:(j + 1) * HID] for j in range(PROJ_BB)], axis=0)
    ns = us * sts + (1.0 - us) * c
    h = jnp.dot(ns, p1_ref[...], preferred_element_type=jnp.float32) + p1b_ref[...]
    h = jnp.maximum(h, 0.0)
    o = jnp.dot(h, p2_ref[...], preferred_element_type=jnp.float32) + p2b_ref[...]
    for j in range(PROJ_BB):
        o_ref[j] = o[j * n:(j + 1) * n]


def _diffuse(s2, x, n_tiles):
    n, c = x.shape
    ct = c // n_tiles
    return pl.pallas_call(
        _diff_kernel,
        grid=(2, n_tiles),
        in_specs=[
            pl.BlockSpec((1, n, n), lambda s, cc: (s, 0, 0)),
            pl.BlockSpec((n, ct), lambda s, cc: (0, cc)),
        ],
        out_specs=pl.BlockSpec((1, 2, n, ct), lambda s, cc: (s, 0, 0, cc)),
        out_shape=jax.ShapeDtypeStruct((2, 2, n, c), jnp.float32),
    )(s2, x)


def kernel(support0, support1, inputs, state, encoder_outputs,
           W_gate, b_gate, W_cand, b_cand, P1, p1b, P2, p2b):
    b, n, in_dim = inputs.shape
    t_len = encoder_outputs.shape[1]
    h = state.shape[2]
    scale = math.sqrt(float(n * h))
    n_chunks = b // PROJ_BB

    s2 = jnp.stack([support0, support1]).astype(jnp.bfloat16)

    wt = pl.pallas_call(
        functools.partial(_att_kernel, t_len=t_len, scale=scale),
        grid=(b // ATT_BB,),
        in_specs=[
            pl.BlockSpec((ATT_BB, n, h), lambda i: (i, 0, 0)),
            pl.BlockSpec((ATT_BB, t_len, n, h), lambda i: (i, 0, 0, 0)),
        ],
        out_specs=pl.BlockSpec((n, ATT_BB * h), lambda i: (0, i)),
        out_shape=jax.ShapeDtypeStruct((n, b * h), jnp.float32),
    )(state, encoder_outputs)

    st = state.transpose(1, 0, 2).reshape(n, b * h)
    i0 = inputs.transpose(1, 0, 2).reshape(n, b * in_dim)

    di = _diffuse(s2, i0, 1)
    dw = _diffuse(s2, wt, 2)
    ds = _diffuse(s2, st, 2)

    cin = PROJ_BB * in_dim
    i0r = i0.reshape(n, n_chunks, cin).transpose(1, 0, 2)
    dir2 = di.reshape(2, 2, n, n_chunks, cin).transpose(3, 0, 1, 2, 4)

    in_size = in_dim + 2 * h
    nm = 2 * MAXSTEP + 1
    wg2 = W_gate.reshape(in_size, nm, 2 * h).transpose(1, 0, 2).reshape(in_size * nm, 2 * h)
    wc2 = W_cand.reshape(in_size, nm, h).transpose(1, 0, 2).reshape(in_size * nm, h)

    full = lambda shp: pl.BlockSpec(shp, lambda i: tuple(0 for _ in shp))
    cw = PROJ_BB * h
    i0_spec = pl.BlockSpec((1, n, cin), lambda i: (i, 0, 0))
    di_spec = pl.BlockSpec((1, 2, 2, n, cin), lambda i: (i, 0, 0, 0, 0))
    col_spec = pl.BlockSpec((n, cw), lambda i: (0, i))
    d_spec = pl.BlockSpec((2, 2, n, cw), lambda i: (0, 0, 0, i))
    u_spec = pl.BlockSpec((PROJ_BB, n, h), lambda i: (i, 0, 0))

    rst, u = pl.pallas_call(
        functools.partial(_gate_kernel, in_dim=in_dim, n=n),
        grid=(n_chunks,),
        in_specs=[i0_spec, di_spec, col_spec, d_spec, col_spec, d_spec,
                  full((in_size * nm, 2 * h)), full((1, 2 * h))],
        out_specs=[col_spec, u_spec],
        out_shape=[jax.ShapeDtypeStruct((n, b * h), jnp.float32),
                   jax.ShapeDtypeStruct((b, n, h), jnp.float32)],
    )(i0r, dir2, wt, dw, st, ds, wg2, b_gate.reshape(1, 2 * h))

    dr = _diffuse(s2, rst, 2)

    out = pl.pallas_call(
        functools.partial(_final_kernel, in_dim=in_dim, n=n),
        grid=(n_chunks,),
        in_specs=[i0_spec, di_spec, col_spec, d_spec, col_spec, d_spec,
                  u_spec, col_spec,
                  full((in_size * nm, h)), full((1, h)),
                  full((h, h)), full((1, h)), full((h, 1)), full((1, 1))],
        out_specs=pl.BlockSpec((PROJ_BB, n, 1), lambda i: (i, 0, 0)),
        out_shape=jax.ShapeDtypeStruct((b, n, 1), jnp.float32),
    )(i0r, dir2, wt, dw, rst, dr, u, st, wc2,
      b_cand.reshape(1, h), P1, p1b.reshape(1, h), P2, p2b.reshape(1, 1))

    return out.reshape(b, n)


# speedup vs baseline: 1.8393x; 1.1199x over previous
import functools
import math

import jax
import jax.numpy as jnp
from jax.experimental import pallas as pl

HID = 64
MAXSTEP = 2
ATT_BB = 2
PROJ_BB = 4
BF = jnp.bfloat16
F32 = jnp.float32


def _att_kernel(state_ref, enc_ref, folde_ref, exp12_ref, foldh_ref, out_ref,
                *, t_len, scale):
    for j in range(ATT_BB):
        ec = jnp.concatenate(
            [enc_ref[j, t].astype(BF) for t in range(t_len)], axis=1)
        stb = state_ref[j].astype(BF)
        stt = jnp.concatenate([stb] * t_len, axis=1)
        m = stt * ec
        e = jnp.dot(m, folde_ref[...], preferred_element_type=F32)
        e = e * (1.0 / scale)
        mx = jnp.max(e, axis=1, keepdims=True)
        w = jnp.exp(e - mx)
        den = jnp.sum(w, axis=1, keepdims=True)
        p = (w / den).astype(BF)
        pex = jnp.dot(p, exp12_ref[...], preferred_element_type=F32).astype(BF)
        wc = pex * ec
        wtd = jnp.dot(wc, foldh_ref[...], preferred_element_type=F32)
        out_ref[:, j * HID:(j + 1) * HID] = wtd.astype(BF)


def _diff_kernel(s_ref, x_ref, d_ref):
    s = s_ref[0]
    x1 = jnp.dot(s, x_ref[...], preferred_element_type=F32)
    x1b = x1.astype(BF)
    x2 = jnp.dot(s, x1b, preferred_element_type=F32)
    d_ref[0, 0] = x1b
    d_ref[0, 1] = x2.astype(BF)


def _pieces(i0_ref, di_ref, wt_ref, dw_ref, st_ref, ds_ref, j, in_dim):
    cols = [i0_ref[0][:, j * in_dim:(j + 1) * in_dim],
            wt_ref[:, j * HID:(j + 1) * HID],
            st_ref[:, j * HID:(j + 1) * HID]]
    for s in range(2):
        for k in range(2):
            cols += [di_ref[0, s, k][:, j * in_dim:(j + 1) * in_dim],
                     dw_ref[s, k][:, j * HID:(j + 1) * HID],
                     ds_ref[s, k][:, j * HID:(j + 1) * HID]]
    return jnp.concatenate(cols, axis=1)


def _gate_kernel(i0_ref, di_ref, wt_ref, dw_ref, st_ref, ds_ref,
                 wg_ref, bg_ref, rst_ref, u_ref, *, in_dim, n):
    xs = [_pieces(i0_ref, di_ref, wt_ref, dw_ref, st_ref, ds_ref, j, in_dim)
          for j in range(PROJ_BB)]
    x = jnp.concatenate(xs, axis=0)
    g = jnp.dot(x, wg_ref[...], preferred_element_type=F32) + bg_ref[...]
    g = jax.nn.sigmoid(g)
    for j in range(PROJ_BB):
        gj = g[j * n:(j + 1) * n]
        stj = st_ref[:, j * HID:(j + 1) * HID]
        rst_ref[:, j * HID:(j + 1) * HID] = (gj[:, :HID] * stj.astype(F32)).astype(BF)
        u_ref[j] = gj[:, HID:]


def _final_kernel(i0_ref, di_ref, wt_ref, dw_ref, rst_ref, dr_ref,
                  u_ref, st_ref, wc_ref, bc_ref, p1_ref, p1b_ref,
                  p2_ref, p2b_ref, o_ref, *, in_dim, n):
    xs = [_pieces(i0_ref, di_ref, wt_ref, dw_ref, rst_ref, dr_ref, j, in_dim)
          for j in range(PROJ_BB)]
    x = jnp.concatenate(xs, axis=0)
    c = jnp.dot(x, wc_ref[...], preferred_element_type=F32) + bc_ref[...]
    c = jnp.tanh(c)
    us = jnp.concatenate([u_ref[j] for j in range(PROJ_BB)], axis=0)
    sts = st_ref[...].reshape(PROJ_BB * n, HID)
    ns = us * sts + (1.0 - us) * c
    h = jnp.dot(ns.astype(BF), p1_ref[...], preferred_element_type=F32) + p1b_ref[...]
    h = jnp.maximum(h, 0.0)
    o = jnp.dot(h.astype(BF), p2_ref[...], preferred_element_type=F32) + p2b_ref[...]
    for j in range(PROJ_BB):
        o_ref[j] = o[j * n:(j + 1) * n]


def _diffuse(s2, x, n_tiles):
    n, c = x.shape
    ct = c // n_tiles
    return pl.pallas_call(
        _diff_kernel,
        grid=(2, n_tiles),
        in_specs=[
            pl.BlockSpec((1, n, n), lambda s, cc: (s, 0, 0)),
            pl.BlockSpec((n, ct), lambda s, cc: (0, cc)),
        ],
        out_specs=pl.BlockSpec((1, 2, n, ct), lambda s, cc: (s, 0, 0, cc)),
        out_shape=jax.ShapeDtypeStruct((2, 2, n, c), BF),
    )(s2, x)


def kernel(support0, support1, inputs, state, encoder_outputs,
           W_gate, b_gate, W_cand, b_cand, P1, p1b, P2, p2b):
    b, n, in_dim = inputs.shape
    t_len = encoder_outputs.shape[1]
    h = state.shape[2]
    scale = math.sqrt(float(n * h))
    n_chunks = b // PROJ_BB

    s2 = jnp.stack([support0, support1]).astype(BF)

    th = t_len * h
    lane = jnp.arange(th)
    folde = (lane[:, None] // h == jnp.arange(t_len)[None, :]).astype(BF)
    foldh = (lane[:, None] % h == jnp.arange(h)[None, :]).astype(BF)
    exp12 = folde.T

    wt = pl.pallas_call(
        functools.partial(_att_kernel, t_len=t_len, scale=scale),
        grid=(b // ATT_BB,),
        in_specs=[
            pl.BlockSpec((ATT_BB, n, h), lambda i: (i, 0, 0)),
            pl.BlockSpec((ATT_BB, t_len, n, h), lambda i: (i, 0, 0, 0)),
            pl.BlockSpec((th, t_len), lambda i: (0, 0)),
            pl.BlockSpec((t_len, th), lambda i: (0, 0)),
            pl.BlockSpec((th, h), lambda i: (0, 0)),
        ],
        out_specs=pl.BlockSpec((n, ATT_BB * h), lambda i: (0, i)),
        out_shape=jax.ShapeDtypeStruct((n, b * h), BF),
    )(state, encoder_outputs, folde, exp12, foldh)

    st = state.transpose(1, 0, 2).reshape(n, b * h).astype(BF)
    i0 = inputs.transpose(1, 0, 2).reshape(n, b * in_dim).astype(BF)

    di = _diffuse(s2, i0, 1)
    dw = _diffuse(s2, wt, 2)
    ds = _diffuse(s2, st, 2)

    cin = PROJ_BB * in_dim
    i0r = i0.reshape(n, n_chunks, cin).transpose(1, 0, 2)
    dir2 = di.reshape(2, 2, n, n_chunks, cin).transpose(3, 0, 1, 2, 4)

    in_size = in_dim + 2 * h
    nm = 2 * MAXSTEP + 1
    wg2 = W_gate.reshape(in_size, nm, 2 * h).transpose(1, 0, 2).reshape(in_size * nm, 2 * h).astype(BF)
    wc2 = W_cand.reshape(in_size, nm, h).transpose(1, 0, 2).reshape(in_size * nm, h).astype(BF)

    full = lambda shp: pl.BlockSpec(shp, lambda i: tuple(0 for _ in shp))
    cw = PROJ_BB * h
    i0_spec = pl.BlockSpec((1, n, cin), lambda i: (i, 0, 0))
    di_spec = pl.BlockSpec((1, 2, 2, n, cin), lambda i: (i, 0, 0, 0, 0))
    col_spec = pl.BlockSpec((n, cw), lambda i: (0, i))
    d_spec = pl.BlockSpec((2, 2, n, cw), lambda i: (0, 0, 0, i))
    u_spec = pl.BlockSpec((PROJ_BB, n, h), lambda i: (i, 0, 0))

    rst, u = pl.pallas_call(
        functools.partial(_gate_kernel, in_dim=in_dim, n=n),
        grid=(n_chunks,),
        in_specs=[i0_spec, di_spec, col_spec, d_spec, col_spec, d_spec,
                  full((in_size * nm, 2 * h)), full((1, 2 * h))],
        out_specs=[col_spec, u_spec],
        out_shape=[jax.ShapeDtypeStruct((n, b * h), BF),
                   jax.ShapeDtypeStruct((b, n, h), F32)],
    )(i0r, dir2, wt, dw, st, ds, wg2, b_gate.reshape(1, 2 * h))

    dr = _diffuse(s2, rst, 2)

    out = pl.pallas_call(
        functools.partial(_final_kernel, in_dim=in_dim, n=n),
        grid=(n_chunks,),
        in_specs=[i0_spec, di_spec, col_spec, d_spec, col_spec, d_spec,
                  u_spec, u_spec,
                  full((in_size * nm, h)), full((1, h)),
                  full((h, h)), full((1, h)), full((h, 1)), full((1, 1))],
        out_specs=pl.BlockSpec((PROJ_BB, n, 1), lambda i: (i, 0, 0)),
        out_shape=jax.ShapeDtypeStruct((b, n, 1), F32),
    )(i0r, dir2, wt, dw, rst, dr, u, state, wc2,
      b_cand.reshape(1, h), P1.astype(BF), p1b.reshape(1, h),
      P2.astype(BF), p2b.reshape(1, 1))

    return out.reshape(b, n)


# speedup vs baseline: 1.8573x; 1.0098x over previous
import functools
import math

import jax
import jax.numpy as jnp
from jax.experimental import pallas as pl

HID = 64
MAXSTEP = 2
BB = 2
BF = jnp.bfloat16
F32 = jnp.float32


def _attention(state_ref, enc_ref, folde_ref, exp12_ref, foldh_ref, j,
               t_len, scale):
    ec = jnp.concatenate(
        [enc_ref[j, t].astype(BF) for t in range(t_len)], axis=1)
    stb = state_ref[j].astype(BF)
    stt = jnp.concatenate([stb] * t_len, axis=1)
    m = stt * ec
    e = jnp.dot(m, folde_ref[...], preferred_element_type=F32)
    e = e * (1.0 / scale)
    mx = jnp.max(e, axis=1, keepdims=True)
    w = jnp.exp(e - mx)
    den = jnp.sum(w, axis=1, keepdims=True)
    p = (w / den).astype(BF)
    pex = jnp.dot(p, exp12_ref[...], preferred_element_type=F32).astype(BF)
    wc = pex * ec
    wtd = jnp.dot(wc, foldh_ref[...], preferred_element_type=F32)
    return wtd.astype(BF)


def _mega_kernel(state_ref, enc_ref, inp_ref, s2_ref,
                 folde_ref, exp12_ref, foldh_ref,
                 wg_ref, bg_ref, wc_ref, bc_ref,
                 p1_ref, p1b_ref, p2_ref, p2b_ref, o_ref,
                 *, t_len, scale, in_dim, n):
    wts = [_attention(state_ref, enc_ref, folde_ref, exp12_ref, foldh_ref,
                      j, t_len, scale) for j in range(BB)]

    ibs = [inp_ref[j].astype(BF) for j in range(BB)]
    stbs = [state_ref[j].astype(BF) for j in range(BB)]
    xc = jnp.concatenate(ibs + wts + stbs, axis=1)

    chains = []
    for s in range(2):
        x1 = jnp.dot(s2_ref[s], xc, preferred_element_type=F32).astype(BF)
        x2 = jnp.dot(s2_ref[s], x1, preferred_element_type=F32).astype(BF)
        chains += [x1, x2]

    oi = lambda j: j * in_dim
    ow = lambda j: BB * in_dim + j * HID
    osn = lambda j: BB * in_dim + BB * HID + j * HID

    def trio(arr, j):
        return [arr[:, oi(j):oi(j) + in_dim],
                arr[:, ow(j):ow(j) + HID],
                arr[:, osn(j):osn(j) + HID]]

    xs = []
    for j in range(BB):
        cols = trio(xc, j)
        for ch in chains:
            cols += trio(ch, j)
        xs.append(jnp.concatenate(cols, axis=1))
    xg = jnp.concatenate(xs, axis=0)
    g = jnp.dot(xg, wg_ref[...], preferred_element_type=F32) + bg_ref[...]
    g = jax.nn.sigmoid(g)

    rss = [(g[j * n:(j + 1) * n, :HID] * state_ref[j]).astype(BF)
           for j in range(BB)]
    xr = jnp.concatenate(rss, axis=1)
    rchains = []
    for s in range(2):
        x1 = jnp.dot(s2_ref[s], xr, preferred_element_type=F32).astype(BF)
        x2 = jnp.dot(s2_ref[s], x1, preferred_element_type=F32).astype(BF)
        rchains += [x1, x2]

    xs = []
    for j in range(BB):
        cols = trio(xc, j)[:2] + [xr[:, j * HID:(j + 1) * HID]]
        for ch, rch in zip(chains, rchains):
            cols += trio(ch, j)[:2] + [rch[:, j * HID:(j + 1) * HID]]
        xs.append(jnp.concatenate(cols, axis=1))
    xcand = jnp.concatenate(xs, axis=0)
    c = jnp.dot(xcand, wc_ref[...], preferred_element_type=F32) + bc_ref[...]
    c = jnp.tanh(c)

    u = g[:, HID:]
    sts = state_ref[...].reshape(BB * n, HID)
    ns = u * sts + (1.0 - u) * c
    h1 = jnp.dot(ns.astype(BF), p1_ref[...], preferred_element_type=F32) + p1b_ref[...]
    h1 = jnp.maximum(h1, 0.0)
    o = jnp.dot(h1.astype(BF), p2_ref[...], preferred_element_type=F32) + p2b_ref[...]
    for j in range(BB):
        o_ref[j] = o[j * n:(j + 1) * n]


def kernel(support0, support1, inputs, state, encoder_outputs,
           W_gate, b_gate, W_cand, b_cand, P1, p1b, P2, p2b):
    b, n, in_dim = inputs.shape
    t_len = encoder_outputs.shape[1]
    h = state.shape[2]
    scale = math.sqrt(float(n * h))

    s2 = jnp.stack([support0, support1]).astype(BF)

    th = t_len * h
    lane = jnp.arange(th)
    folde = (lane[:, None] // h == jnp.arange(t_len)[None, :]).astype(BF)
    foldh = (lane[:, None] % h == jnp.arange(h)[None, :]).astype(BF)
    exp12 = folde.T

    in_size = in_dim + 2 * h
    nm = 2 * MAXSTEP + 1
    wg2 = W_gate.reshape(in_size, nm, 2 * h).transpose(1, 0, 2).reshape(in_size * nm, 2 * h).astype(BF)
    wc2 = W_cand.reshape(in_size, nm, h).transpose(1, 0, 2).reshape(in_size * nm, h).astype(BF)

    full = lambda shp: pl.BlockSpec(shp, lambda i: tuple(0 for _ in shp))

    out = pl.pallas_call(
        functools.partial(_mega_kernel, t_len=t_len, scale=scale,
                          in_dim=in_dim, n=n),
        grid=(b // BB,),
        in_specs=[
            pl.BlockSpec((BB, n, h), lambda i: (i, 0, 0)),
            pl.BlockSpec((BB, t_len, n, h), lambda i: (i, 0, 0, 0)),
            pl.BlockSpec((BB, n, in_dim), lambda i: (i, 0, 0)),
            full((2, n, n)),
            full((th, t_len)), full((t_len, th)), full((th, h)),
            full((in_size * nm, 2 * h)), full((1, 2 * h)),
            full((in_size * nm, h)), full((1, h)),
            full((h, h)), full((1, h)), full((h, 1)), full((1, 1)),
        ],
        out_specs=pl.BlockSpec((BB, n, 1), lambda i: (i, 0, 0)),
        out_shape=jax.ShapeDtypeStruct((b, n, 1), F32),
    )(state, encoder_outputs, inputs, s2, folde, exp12, foldh,
      wg2, b_gate.reshape(1, 2 * h), wc2, b_cand.reshape(1, h),
      P1.astype(BF), p1b.reshape(1, h), P2.astype(BF), p2b.reshape(1, 1))

    return out.reshape(b, n)
